# Initial kernel scaffold; baseline (speedup 1.0000x reference)
import jax
import jax.numpy as jnp
from jax.experimental import pallas as pl


def kernel(atom_ids, hybrid_ids, node_continuous, atom_table, hybrid_table, Wc, bc, Wf, bf, Wo, bo):
    raise NotImplementedError("write your pallas kernel here")


# speedup vs baseline: 2.9253x; 2.9253x over previous
import functools

import jax
import jax.numpy as jnp
from jax import lax
from jax.experimental import pallas as pl
from jax.experimental.pallas import tpu as pltpu

ATOM_VOCAB = 128
HYBRID_VOCAB = 8
ATOM_DIM = 32
HYBRID_DIM = 16
HIDDEN = 64


def _fused_body(aid_ref, hid_ref, x_ref, at_ref, ht_ref, wc_ref, bc_ref,
                wf_ref, bf_ref, wo_ref, bo_ref, o_ref,
                ta_s, th_s, w1_s, b1_s, *, bt):
    pid = pl.program_id(0)

    @pl.when(pid == 0)
    def _():
        wf_a = wf_ref[0:ATOM_DIM, :]
        wf_h = wf_ref[ATOM_DIM:ATOM_DIM + HYBRID_DIM, :]
        wf_c = wf_ref[ATOM_DIM + HYBRID_DIM:, :]
        ta_s[...] = jnp.dot(at_ref[...], wf_a, preferred_element_type=jnp.float32)
        th_s[...] = jnp.dot(ht_ref[...], wf_h, preferred_element_type=jnp.float32)
        w1_s[...] = jnp.dot(wc_ref[...], wf_c, preferred_element_type=jnp.float32)
        b1_s[...] = jnp.dot(bc_ref[...], wf_c, preferred_element_type=jnp.float32) + bf_ref[...]

    aid = aid_ref[...]
    hid = hid_ref[...]
    oh_a = (aid == lax.broadcasted_iota(jnp.int32, (bt, ATOM_VOCAB), 1)).astype(jnp.float32)
    oh_h = (hid == lax.broadcasted_iota(jnp.int32, (bt, HYBRID_VOCAB), 1)).astype(jnp.float32)
    h1 = (jnp.dot(oh_a, ta_s[...], preferred_element_type=jnp.float32)
          + jnp.dot(oh_h, th_s[...], preferred_element_type=jnp.float32)
          + jnp.dot(x_ref[...], w1_s[...], preferred_element_type=jnp.float32)
          + b1_s[...])
    h = jax.nn.gelu(h1, approximate=True)
    o_ref[...] = jnp.dot(h, wo_ref[...], preferred_element_type=jnp.float32) + bo_ref[...]


def kernel(atom_ids, hybrid_ids, node_continuous, atom_table, hybrid_table,
           Wc, bc, Wf, bf, Wo, bo):
    B, A = atom_ids.shape
    T = B * A
    BT = 2048
    assert T % BT == 0
    grid = (T // BT,)
    cont_in = node_continuous.shape[-1]

    aid2 = atom_ids.reshape(T, 1)
    hid2 = hybrid_ids.reshape(T, 1)
    x2 = node_continuous.reshape(T, cont_in)
    bc2 = bc.reshape(1, -1)
    bf2 = bf.reshape(1, -1)
    bo2 = bo.reshape(1, -1)

    rep = lambda shape: pl.BlockSpec(shape, lambda i: (0,) * len(shape))
    out = pl.pallas_call(
        functools.partial(_fused_body, bt=BT),
        grid=grid,
        in_specs=[
            pl.BlockSpec((BT, 1), lambda i: (i, 0)),
            pl.BlockSpec((BT, 1), lambda i: (i, 0)),
            pl.BlockSpec((BT, cont_in), lambda i: (i, 0)),
            rep(atom_table.shape),
            rep(hybrid_table.shape),
            rep(Wc.shape),
            rep(bc2.shape),
            rep(Wf.shape),
            rep(bf2.shape),
            rep(Wo.shape),
            rep(bo2.shape),
        ],
        out_specs=pl.BlockSpec((BT, HIDDEN), lambda i: (i, 0)),
        out_shape=jax.ShapeDtypeStruct((T, HIDDEN), jnp.float32),
        scratch_shapes=[
            pltpu.VMEM((ATOM_VOCAB, HIDDEN), jnp.float32),
            pltpu.VMEM((HYBRID_VOCAB, HIDDEN), jnp.float32),
            pltpu.VMEM((cont_in, HIDDEN), jnp.float32),
            pltpu.VMEM((1, HIDDEN), jnp.float32),
        ],
    )(aid2, hid2, x2, atom_table, hybrid_table, Wc, bc2, Wf, bf2, Wo, bo2)
    return out.reshape(B, A, HIDDEN)


# speedup vs baseline: 4.1191x; 1.4081x over previous
import functools

import jax
import jax.numpy as jnp
from jax import lax
from jax.experimental import pallas as pl
from jax.experimental.pallas import tpu as pltpu

ATOM_VOCAB = 128
HYBRID_VOCAB = 8
ATOM_DIM = 32
HYBRID_DIM = 16
HIDDEN = 64


def _dg(a, b, dims):
    return lax.dot_general(a, b, (dims, ((), ())),
                           preferred_element_type=jnp.float32)


def _fused_body(ids_ref, x_ref, at_ref, ht_ref, wc_ref, bc_ref,
                wf_ref, bfc_ref, wo_ref, bo_ref, o_ref,
                ta_s, th_s, w1t_s, b1c_s, *, bt):
    @pl.when(pl.program_id(0) == 0)
    def _():
        wf_a = wf_ref[0:ATOM_DIM, :]
        wf_h = wf_ref[ATOM_DIM:ATOM_DIM + HYBRID_DIM, :]
        wf_c = wf_ref[ATOM_DIM + HYBRID_DIM:, :]
        ta_s[...] = _dg(wf_a, at_ref[...], ((0,), (1,)))
        th_s[...] = _dg(wf_h, ht_ref[...], ((0,), (1,)))
        w1t_s[...] = _dg(wf_c, wc_ref[...], ((0,), (1,)))
        b1c_s[...] = _dg(wf_c, bc_ref[...], ((0,), (1,))) + bfc_ref[...]

    aid = ids_ref[0, 0]
    hid = ids_ref[1, 0]
    oh_a = (aid == lax.broadcasted_iota(jnp.int32, (ATOM_VOCAB, bt), 0)).astype(jnp.float32)
    oh_h = (hid == lax.broadcasted_iota(jnp.int32, (HYBRID_VOCAB, bt), 0)).astype(jnp.float32)
    h1t = (_dg(ta_s[...], oh_a, ((1,), (0,)))
           + _dg(th_s[...], oh_h, ((1,), (0,)))
           + _dg(w1t_s[...], x_ref[...], ((1,), (1,)))
           + b1c_s[...])
    h = jax.nn.gelu(h1t, approximate=True)
    o_ref[...] = _dg(h, wo_ref[...], ((0,), (0,))) + bo_ref[...]


def kernel(atom_ids, hybrid_ids, node_continuous, atom_table, hybrid_table,
           Wc, bc, Wf, bf, Wo, bo):
    B, A = atom_ids.shape
    T = B * A
    BT = 4096
    assert T % BT == 0
    nb = T // BT
    cont_in = node_continuous.shape[-1]

    ids_both = jnp.stack(
        [atom_ids.reshape(-1), hybrid_ids.reshape(-1)]).reshape(2, nb, 1, BT)
    x2 = node_continuous.reshape(T, cont_in)
    bc2 = bc.reshape(1, -1)
    bfc = bf.reshape(-1, 1)
    bo2 = bo.reshape(1, -1)

    rep = lambda shape: pl.BlockSpec(shape, lambda i: (0,) * len(shape))
    out = pl.pallas_call(
        functools.partial(_fused_body, bt=BT),
        grid=(nb,),
        in_specs=[
            pl.BlockSpec((2, 1, 1, BT), lambda i: (0, i, 0, 0)),
            pl.BlockSpec((BT, cont_in), lambda i: (i, 0)),
            rep(atom_table.shape),
            rep(hybrid_table.shape),
            rep(Wc.shape),
            rep(bc2.shape),
            rep(Wf.shape),
            rep(bfc.shape),
            rep(Wo.shape),
            rep(bo2.shape),
        ],
        out_specs=pl.BlockSpec((BT, HIDDEN), lambda i: (i, 0)),
        out_shape=jax.ShapeDtypeStruct((T, HIDDEN), jnp.float32),
        scratch_shapes=[
            pltpu.VMEM((HIDDEN, ATOM_VOCAB), jnp.float32),
            pltpu.VMEM((HIDDEN, HYBRID_VOCAB), jnp.float32),
            pltpu.VMEM((HIDDEN, cont_in), jnp.float32),
            pltpu.VMEM((HIDDEN, 1), jnp.float32),
        ],
    )(ids_both, x2, atom_table, hybrid_table, Wc, bc2, Wf, bfc, Wo, bo2)
    return out.reshape(B, A, HIDDEN)


# speedup vs baseline: 15.1395x; 3.6754x over previous
import functools

import jax
import jax.numpy as jnp
from jax import lax
from jax.experimental import pallas as pl
from jax.experimental.pallas import tpu as pltpu

ATOM_VOCAB = 128
HYBRID_VOCAB = 8
ATOM_DIM = 32
HYBRID_DIM = 16
HIDDEN = 64


def _dg(a, b, dims):
    return lax.dot_general(a, b, (dims, ((), ())),
                           preferred_element_type=jnp.float32)


def _fused_body(aT_ref, hT_ref, xT_ref, at_ref, ht_ref, wc_ref, bc_ref,
                wf_ref, bfc_ref, wo_ref, boc_ref, o_ref,
                ta_s, th_s, w1t_s, b1c_s, *, bt, na):
    @pl.when(pl.program_id(0) == 0)
    def _():
        wf_a = wf_ref[0:ATOM_DIM, :]
        wf_h = wf_ref[ATOM_DIM:ATOM_DIM + HYBRID_DIM, :]
        wf_c = wf_ref[ATOM_DIM + HYBRID_DIM:, :]
        ta_s[...] = _dg(wf_a, at_ref[...], ((0,), (1,)))
        th_s[...] = _dg(wf_h, ht_ref[...], ((0,), (1,)))
        w1t_s[...] = _dg(wf_c, wc_ref[...], ((0,), (1,)))
        b1c_s[...] = _dg(wf_c, bc_ref[...], ((0,), (1,))) + bfc_ref[...]

    ta = ta_s[...]
    th = th_s[...]
    w1t = w1t_s[...]
    b1c = b1c_s[...]
    wo = wo_ref[...]
    boc = boc_ref[...]
    for a in range(na):
        aid = aT_ref[a:a + 1, :]
        hid = hT_ref[a:a + 1, :]
        oh_a = (aid == lax.broadcasted_iota(jnp.int32, (ATOM_VOCAB, bt), 0)).astype(jnp.float32)
        oh_h = (hid == lax.broadcasted_iota(jnp.int32, (HYBRID_VOCAB, bt), 0)).astype(jnp.float32)
        h1t = (_dg(ta, oh_a, ((1,), (0,)))
               + _dg(th, oh_h, ((1,), (0,)))
               + _dg(w1t, xT_ref[a], ((1,), (0,)))
               + b1c)
        h = jax.nn.gelu(h1t, approximate=True)
        o_ref[a] = _dg(wo, h, ((0,), (0,))) + boc


def kernel(atom_ids, hybrid_ids, node_continuous, atom_table, hybrid_table,
           Wc, bc, Wf, bf, Wo, bo):
    B, A = atom_ids.shape
    BT = 512
    assert B % BT == 0
    nb = B // BT
    cont_in = node_continuous.shape[-1]

    aT = atom_ids.T
    hT = hybrid_ids.T
    xT = jnp.transpose(node_continuous, (1, 2, 0))
    bc2 = bc.reshape(1, -1)
    bfc = bf.reshape(-1, 1)
    boc = bo.reshape(-1, 1)

    rep = lambda shape: pl.BlockSpec(shape, lambda i: (0,) * len(shape))
    outT = pl.pallas_call(
        functools.partial(_fused_body, bt=BT, na=A),
        grid=(nb,),
        in_specs=[
            pl.BlockSpec((A, BT), lambda i: (0, i)),
            pl.BlockSpec((A, BT), lambda i: (0, i)),
            pl.BlockSpec((A, cont_in, BT), lambda i: (0, 0, i)),
            rep(atom_table.shape),
            rep(hybrid_table.shape),
            rep(Wc.shape),
            rep(bc2.shape),
            rep(Wf.shape),
            rep(bfc.shape),
            rep(Wo.shape),
            rep(boc.shape),
        ],
        out_specs=pl.BlockSpec((A, HIDDEN, BT), lambda i: (0, 0, i)),
        out_shape=jax.ShapeDtypeStruct((A, HIDDEN, B), jnp.float32),
        scratch_shapes=[
            pltpu.VMEM((HIDDEN, ATOM_VOCAB), jnp.float32),
            pltpu.VMEM((HIDDEN, HYBRID_VOCAB), jnp.float32),
            pltpu.VMEM((HIDDEN, cont_in), jnp.float32),
            pltpu.VMEM((HIDDEN, 1), jnp.float32),
        ],
    )(aT, hT, xT, atom_table, hybrid_table, Wc, bc2, Wf, bfc, Wo, boc)
    return jnp.transpose(outT, (2, 0, 1))


# speedup vs baseline: 20.7282x; 1.3691x over previous
import functools

import jax
import jax.numpy as jnp
from jax import lax
from jax.experimental import pallas as pl
from jax.experimental.pallas import tpu as pltpu

ATOM_VOCAB = 128
HYBRID_VOCAB = 8
ATOM_DIM = 32
HYBRID_DIM = 16
HIDDEN = 64


def _dg(a, b, dims):
    return lax.dot_general(a, b, (dims, ((), ())),
                           preferred_element_type=jnp.float32)


def _fused_body(aT_ref, hT_ref, xT_ref, at_ref, ht_ref, wc_ref, bc_ref,
                wf_ref, bfc_ref, wo_ref, boc_ref, o_ref,
                ta_s, th_s, w1t_s, b1c_s, *, bt, na):
    @pl.when(pl.program_id(0) == 0)
    def _():
        wf_a = wf_ref[0:ATOM_DIM, :]
        wf_h = wf_ref[ATOM_DIM:ATOM_DIM + HYBRID_DIM, :]
        wf_c = wf_ref[ATOM_DIM + HYBRID_DIM:, :]
        ta_s[...] = _dg(wf_a, at_ref[...], ((0,), (1,))).astype(jnp.bfloat16)
        th_s[...] = _dg(wf_h, ht_ref[...], ((0,), (1,))).astype(jnp.bfloat16)
        w1t_s[...] = _dg(wf_c, wc_ref[...], ((0,), (1,)))
        b1c_s[...] = _dg(wf_c, bc_ref[...], ((0,), (1,))) + bfc_ref[...]

    ta = ta_s[...]
    th = th_s[...]
    w1t = w1t_s[...]
    b1c = b1c_s[...]
    wo = wo_ref[...]
    boc = boc_ref[...]
    for a in range(na):
        aid = aT_ref[a:a + 1, :]
        hid = hT_ref[a:a + 1, :]
        oh_a = (aid == lax.broadcasted_iota(jnp.int32, (ATOM_VOCAB, bt), 0)).astype(jnp.bfloat16)
        oh_h = (hid == lax.broadcasted_iota(jnp.int32, (HYBRID_VOCAB, bt), 0)).astype(jnp.bfloat16)
        h1t = (_dg(ta, oh_a, ((1,), (0,)))
               + _dg(th, oh_h, ((1,), (0,)))
               + _dg(w1t, xT_ref[a], ((1,), (0,)))
               + b1c)
        h = jax.nn.gelu(h1t, approximate=True)
        o_ref[a] = _dg(wo, h, ((0,), (0,))) + boc


def kernel(atom_ids, hybrid_ids, node_continuous, atom_table, hybrid_table,
           Wc, bc, Wf, bf, Wo, bo):
    B, A = atom_ids.shape
    BT = 1024
    assert B % BT == 0
    nb = B // BT
    cont_in = node_continuous.shape[-1]

    aT = atom_ids.T
    hT = hybrid_ids.T
    xT = jnp.transpose(node_continuous, (1, 2, 0))
    bc2 = bc.reshape(1, -1)
    bfc = bf.reshape(-1, 1)
    boc = bo.reshape(-1, 1)

    rep = lambda shape: pl.BlockSpec(shape, lambda i: (0,) * len(shape))
    outT = pl.pallas_call(
        functools.partial(_fused_body, bt=BT, na=A),
        grid=(nb,),
        in_specs=[
            pl.BlockSpec((A, BT), lambda i: (0, i)),
            pl.BlockSpec((A, BT), lambda i: (0, i)),
            pl.BlockSpec((A, cont_in, BT), lambda i: (0, 0, i)),
            rep(atom_table.shape),
            rep(hybrid_table.shape),
            rep(Wc.shape),
            rep(bc2.shape),
            rep(Wf.shape),
            rep(bfc.shape),
            rep(Wo.shape),
            rep(boc.shape),
        ],
        out_specs=pl.BlockSpec((A, HIDDEN, BT), lambda i: (0, 0, i)),
        out_shape=jax.ShapeDtypeStruct((A, HIDDEN, B), jnp.float32),
        scratch_shapes=[
            pltpu.VMEM((HIDDEN, ATOM_VOCAB), jnp.bfloat16),
            pltpu.VMEM((HIDDEN, HYBRID_VOCAB), jnp.bfloat16),
            pltpu.VMEM((HIDDEN, cont_in), jnp.float32),
            pltpu.VMEM((HIDDEN, 1), jnp.float32),
        ],
    )(aT, hT, xT, atom_table, hybrid_table, Wc, bc2, Wf, bfc, Wo, boc)
    return jnp.transpose(outT, (2, 0, 1))


# speedup vs baseline: 22.8067x; 1.1003x over previous
import functools

import jax
import jax.numpy as jnp
from jax import lax
from jax.experimental import pallas as pl
from jax.experimental.pallas import tpu as pltpu

ATOM_VOCAB = 128
HYBRID_VOCAB = 8
ATOM_DIM = 32
HYBRID_DIM = 16
HIDDEN = 64


def _dg(a, b, dims):
    return lax.dot_general(a, b, (dims, ((), ())),
                           preferred_element_type=jnp.float32)


def _fused_body(aT_ref, hT_ref, xT_ref, at_ref, ht_ref, wc_ref, bc_ref,
                wf_ref, bfc_ref, wo_ref, boc_ref, o_ref,
                ta_s, th_s, w1t_s, b1c_s, wob_s, *, bt, na):
    @pl.when(pl.program_id(0) == 0)
    def _():
        wf_a = wf_ref[0:ATOM_DIM, :]
        wf_h = wf_ref[ATOM_DIM:ATOM_DIM + HYBRID_DIM, :]
        wf_c = wf_ref[ATOM_DIM + HYBRID_DIM:, :]
        ta_s[...] = _dg(wf_a, at_ref[...], ((0,), (1,))).astype(jnp.bfloat16)
        th_s[...] = _dg(wf_h, ht_ref[...], ((0,), (1,))).astype(jnp.bfloat16)
        w1t_s[...] = _dg(wf_c, wc_ref[...], ((0,), (1,)))
        b1c_s[...] = _dg(wf_c, bc_ref[...], ((0,), (1,))) + bfc_ref[...]
        wob_s[...] = wo_ref[...].astype(jnp.bfloat16)

    ta = ta_s[...]
    th = th_s[...]
    w1t = w1t_s[...]
    b1c = b1c_s[...]
    wob = wob_s[...]
    boc = boc_ref[...]
    for a in range(na):
        aid = aT_ref[a:a + 1, :]
        hid = hT_ref[a:a + 1, :]
        oh_a = (aid == lax.broadcasted_iota(jnp.int32, (ATOM_VOCAB, bt), 0)).astype(jnp.bfloat16)
        oh_h = (hid == lax.broadcasted_iota(jnp.int32, (HYBRID_VOCAB, bt), 0)).astype(jnp.bfloat16)
        h1t = (_dg(ta, oh_a, ((1,), (0,)))
               + _dg(th, oh_h, ((1,), (0,)))
               + _dg(w1t, xT_ref[a], ((1,), (0,)))
               + b1c)
        h = jax.nn.gelu(h1t.astype(jnp.bfloat16), approximate=True)
        o_ref[a] = _dg(wob, h, ((0,), (0,))) + boc


def kernel(atom_ids, hybrid_ids, node_continuous, atom_table, hybrid_table,
           Wc, bc, Wf, bf, Wo, bo):
    B, A = atom_ids.shape
    BT = 1024
    assert B % BT == 0
    nb = B // BT
    cont_in = node_continuous.shape[-1]

    aT = atom_ids.T
    hT = hybrid_ids.T
    xT = jnp.transpose(node_continuous, (1, 2, 0))
    bc2 = bc.reshape(1, -1)
    bfc = bf.reshape(-1, 1)
    boc = bo.reshape(-1, 1)

    rep = lambda shape: pl.BlockSpec(shape, lambda i: (0,) * len(shape))
    outT = pl.pallas_call(
        functools.partial(_fused_body, bt=BT, na=A),
        grid=(nb,),
        in_specs=[
            pl.BlockSpec((A, BT), lambda i: (0, i)),
            pl.BlockSpec((A, BT), lambda i: (0, i)),
            pl.BlockSpec((A, cont_in, BT), lambda i: (0, 0, i)),
            rep(atom_table.shape),
            rep(hybrid_table.shape),
            rep(Wc.shape),
            rep(bc2.shape),
            rep(Wf.shape),
            rep(bfc.shape),
            rep(Wo.shape),
            rep(boc.shape),
        ],
        out_specs=pl.BlockSpec((A, HIDDEN, BT), lambda i: (0, 0, i)),
        out_shape=jax.ShapeDtypeStruct((A, HIDDEN, B), jnp.float32),
        scratch_shapes=[
            pltpu.VMEM((HIDDEN, ATOM_VOCAB), jnp.bfloat16),
            pltpu.VMEM((HIDDEN, HYBRID_VOCAB), jnp.bfloat16),
            pltpu.VMEM((HIDDEN, cont_in), jnp.float32),
            pltpu.VMEM((HIDDEN, 1), jnp.float32),
            pltpu.VMEM((HIDDEN, HIDDEN), jnp.bfloat16),
        ],
    )(aT, hT, xT, atom_table, hybrid_table, Wc, bc2, Wf, bfc, Wo, boc)
    return jnp.transpose(outT, (2, 0, 1))


# speedup vs baseline: 28.5228x; 1.2506x over previous
import functools

import jax
import jax.numpy as jnp
from jax import lax
from jax.experimental import pallas as pl
from jax.experimental.pallas import tpu as pltpu

ATOM_VOCAB = 128
HYBRID_VOCAB = 8
ATOM_DIM = 32
HYBRID_DIM = 16
HIDDEN = 64


def _dg(a, b, dims):
    return lax.dot_general(a, b, (dims, ((), ())),
                           preferred_element_type=jnp.float32)


def _fused_body(aT_ref, hT_ref, xT_ref, at_ref, ht_ref, wc_ref, bc_ref,
                wf_ref, bfc_ref, wo_ref, boc_ref, o_ref,
                ta_s, th_s, w1t_s, b1c_s, wob_s, *, bt, na):
    @pl.when(jnp.logical_and(pl.program_id(0) == 0, pl.program_id(1) == 0))
    def _():
        wf_a = wf_ref[0:ATOM_DIM, :]
        wf_h = wf_ref[ATOM_DIM:ATOM_DIM + HYBRID_DIM, :]
        wf_c = wf_ref[ATOM_DIM + HYBRID_DIM:, :]
        ta_s[...] = _dg(wf_a, at_ref[...], ((0,), (1,))).astype(jnp.bfloat16)
        th_s[...] = _dg(wf_h, ht_ref[...], ((0,), (1,))).astype(jnp.bfloat16)
        w1t_s[...] = _dg(wf_c, wc_ref[...], ((0,), (1,)))
        b1c_s[...] = _dg(wf_c, bc_ref[...], ((0,), (1,))) + bfc_ref[...]
        wob_s[...] = wo_ref[...].astype(jnp.bfloat16)

    ta = ta_s[...]
    th = th_s[...]
    w1t = w1t_s[...]
    b1c = b1c_s[...]
    wob = wob_s[...]
    boc = boc_ref[...]
    j = pl.program_id(1)
    for a in range(na):
        aid = aT_ref[pl.ds(j * na + a, 1), :]
        hid = hT_ref[pl.ds(j * na + a, 1), :]
        oh_a = (aid == lax.broadcasted_iota(jnp.int32, (ATOM_VOCAB, bt), 0)).astype(jnp.bfloat16)
        oh_h = (hid == lax.broadcasted_iota(jnp.int32, (HYBRID_VOCAB, bt), 0)).astype(jnp.bfloat16)
        h1t = (_dg(ta, oh_a, ((1,), (0,)))
               + _dg(th, oh_h, ((1,), (0,)))
               + _dg(w1t, xT_ref[a], ((1,), (0,)))
               + b1c)
        h = jax.nn.gelu(h1t.astype(jnp.bfloat16), approximate=True)
        o_ref[a] = _dg(wob, h, ((0,), (0,))) + boc


def kernel(atom_ids, hybrid_ids, node_continuous, atom_table, hybrid_table,
           Wc, bc, Wf, bf, Wo, bo):
    B, A = atom_ids.shape
    BT = 2048
    NJ = 2
    assert B % BT == 0 and A % NJ == 0
    nb = B // BT
    na = A // NJ
    cont_in = node_continuous.shape[-1]

    aT = atom_ids.T
    hT = hybrid_ids.T
    xT = jnp.transpose(node_continuous, (1, 2, 0))
    bc2 = bc.reshape(1, -1)
    bfc = bf.reshape(-1, 1)
    boc = bo.reshape(-1, 1)

    rep = lambda shape: pl.BlockSpec(shape, lambda i, j: (0,) * len(shape))
    outT = pl.pallas_call(
        functools.partial(_fused_body, bt=BT, na=na),
        grid=(nb, NJ),
        in_specs=[
            pl.BlockSpec((A, BT), lambda i, j: (0, i)),
            pl.BlockSpec((A, BT), lambda i, j: (0, i)),
            pl.BlockSpec((na, cont_in, BT), lambda i, j: (j, 0, i)),
            rep(atom_table.shape),
            rep(hybrid_table.shape),
            rep(Wc.shape),
            rep(bc2.shape),
            rep(Wf.shape),
            rep(bfc.shape),
            rep(Wo.shape),
            rep(boc.shape),
        ],
        out_specs=pl.BlockSpec((na, HIDDEN, BT), lambda i, j: (j, 0, i)),
        out_shape=jax.ShapeDtypeStruct((A, HIDDEN, B), jnp.float32),
        scratch_shapes=[
            pltpu.VMEM((HIDDEN, ATOM_VOCAB), jnp.bfloat16),
            pltpu.VMEM((HIDDEN, HYBRID_VOCAB), jnp.bfloat16),
            pltpu.VMEM((HIDDEN, cont_in), jnp.float32),
            pltpu.VMEM((HIDDEN, 1), jnp.float32),
            pltpu.VMEM((HIDDEN, HIDDEN), jnp.bfloat16),
        ],
    )(aT, hT, xT, atom_table, hybrid_table, Wc, bc2, Wf, bfc, Wo, boc)
    return jnp.transpose(outT, (2, 0, 1))


# speedup vs baseline: 29.4242x; 1.0316x over previous
import functools

import jax
import jax.numpy as jnp
from jax import lax
from jax.experimental import pallas as pl
from jax.experimental.pallas import tpu as pltpu

ATOM_VOCAB = 128
HYBRID_VOCAB = 8
ATOM_DIM = 32
HYBRID_DIM = 16
HIDDEN = 64


def _dg(a, b, dims):
    return lax.dot_general(a, b, (dims, ((), ())),
                           preferred_element_type=jnp.float32)


def _fused_body(aT_ref, hT_ref, xT_ref, at_ref, ht_ref, wc_ref, bc_ref,
                wf_ref, bfc_ref, wo_ref, boc_ref, o_ref,
                ta_s, th_s, w1t_s, b1c_s, wob_s, *, bt, na):
    @pl.when(jnp.logical_and(pl.program_id(0) == 0, pl.program_id(1) == 0))
    def _():
        wf_a = wf_ref[0:ATOM_DIM, :]
        wf_h = wf_ref[ATOM_DIM:ATOM_DIM + HYBRID_DIM, :]
        wf_c = wf_ref[ATOM_DIM + HYBRID_DIM:, :]
        ta_s[...] = _dg(wf_a, at_ref[...], ((0,), (1,))).astype(jnp.bfloat16)
        th_s[...] = _dg(wf_h, ht_ref[...], ((0,), (1,))).astype(jnp.bfloat16)
        w1t_s[...] = _dg(wf_c, wc_ref[...], ((0,), (1,)))
        b1c_s[...] = _dg(wf_c, bc_ref[...], ((0,), (1,))) + bfc_ref[...]
        wob_s[...] = wo_ref[...].astype(jnp.bfloat16)

    ta = ta_s[...]
    th = th_s[...]
    w1t = w1t_s[...]
    b1c = b1c_s[...]
    wob = wob_s[...]
    boc = boc_ref[...]
    j = pl.program_id(1)
    for a in range(na):
        aid = aT_ref[pl.ds(j * na + a, 1), :]
        hid = hT_ref[pl.ds(j * na + a, 1), :]
        oh_a = (aid == lax.broadcasted_iota(jnp.int32, (ATOM_VOCAB, bt), 0)).astype(jnp.bfloat16)
        oh_h = (hid == lax.broadcasted_iota(jnp.int32, (HYBRID_VOCAB, bt), 0)).astype(jnp.bfloat16)
        h1t = (_dg(ta, oh_a, ((1,), (0,)))
               + _dg(th, oh_h, ((1,), (0,)))
               + _dg(w1t, xT_ref[a], ((1,), (0,)))
               + b1c)
        h = jax.nn.gelu(h1t.astype(jnp.bfloat16), approximate=True)
        o_ref[a] = _dg(wob, h, ((0,), (0,))) + boc


def kernel(atom_ids, hybrid_ids, node_continuous, atom_table, hybrid_table,
           Wc, bc, Wf, bf, Wo, bo):
    B, A = atom_ids.shape
    BT = 2048
    NJ = 5
    assert B % BT == 0 and A % NJ == 0
    nb = B // BT
    na = A // NJ
    cont_in = node_continuous.shape[-1]

    aT = atom_ids.T
    hT = hybrid_ids.T
    xT = jnp.transpose(node_continuous, (1, 2, 0))
    bc2 = bc.reshape(1, -1)
    bfc = bf.reshape(-1, 1)
    boc = bo.reshape(-1, 1)

    rep = lambda shape: pl.BlockSpec(shape, lambda i, j: (0,) * len(shape))
    outT = pl.pallas_call(
        functools.partial(_fused_body, bt=BT, na=na),
        grid=(nb, NJ),
        in_specs=[
            pl.BlockSpec((A, BT), lambda i, j: (0, i)),
            pl.BlockSpec((A, BT), lambda i, j: (0, i)),
            pl.BlockSpec((na, cont_in, BT), lambda i, j: (j, 0, i)),
            rep(atom_table.shape),
            rep(hybrid_table.shape),
            rep(Wc.shape),
            rep(bc2.shape),
            rep(Wf.shape),
            rep(bfc.shape),
            rep(Wo.shape),
            rep(boc.shape),
        ],
        out_specs=pl.BlockSpec((na, HIDDEN, BT), lambda i, j: (j, 0, i)),
        out_shape=jax.ShapeDtypeStruct((A, HIDDEN, B), jnp.float32),
        scratch_shapes=[
            pltpu.VMEM((HIDDEN, ATOM_VOCAB), jnp.bfloat16),
            pltpu.VMEM((HIDDEN, HYBRID_VOCAB), jnp.bfloat16),
            pltpu.VMEM((HIDDEN, cont_in), jnp.float32),
            pltpu.VMEM((HIDDEN, 1), jnp.float32),
            pltpu.VMEM((HIDDEN, HIDDEN), jnp.bfloat16),
        ],
    )(aT, hT, xT, atom_table, hybrid_table, Wc, bc2, Wf, bfc, Wo, boc)
    return jnp.transpose(outT, (2, 0, 1))
